# baseline (speedup 1.0000x reference)
import functools

import jax
import jax.numpy as jnp
from jax import lax
from jax.experimental import pallas as pl
from jax.experimental.pallas import tpu as pltpu
from jax.experimental.pallas import tpu_sc as plsc

N = 10000
E = 320000
D = 128

NC = 2
NS = 16
NW = NC * NS

C = 128
WCH = 80
HALF = WCH // 2
PE = NW * WCH * C
NPAD = 10240
STRIPE = NPAD // NS

BT = 1024

_vector_mesh = plsc.VectorSubcoreMesh(core_axis_name="c", subcore_axis_name="s")


def _sc_degree(dst2d, zeros16):
    @functools.partial(
        pl.kernel,
        out_type=jax.ShapeDtypeStruct((NC, NPAD, 16), jnp.float32),
        mesh=_vector_mesh,
        scratch_types=[
            pltpu.VMEM((WCH, C), jnp.int32),
            pltpu.VMEM((C, 16), jnp.float32),
            pltpu.VMEM_SHARED((NPAD, 16), jnp.float32),
            pltpu.SemaphoreType.DMA,
        ],
    )
    def k(dst_hbm, z_hbm, out_hbm, dstv, ones, hist, sem):
        c = lax.axis_index("c")
        s = lax.axis_index("s")
        wid = s * NC + c

        @pl.loop(0, C)
        def _fill(i):
            ones[i, :] = jnp.full((16,), 1.0, jnp.float32)

        pltpu.sync_copy(z_hbm.at[pl.ds(s * STRIPE, STRIPE)],
                        hist.at[pl.ds(s * STRIPE, STRIPE)])
        pltpu.sync_copy(dst_hbm.at[pl.ds(wid * WCH, WCH)], dstv)
        plsc.subcore_barrier()

        @pl.loop(0, WCH)
        def _acc(j):
            pltpu.sync_copy(ones, hist.at[dstv.at[j]], add=True)

        plsc.subcore_barrier()
        pltpu.sync_copy(hist.at[pl.ds(s * STRIPE, STRIPE)],
                        out_hbm.at[c].at[pl.ds(s * STRIPE, STRIPE)])

    return k(dst2d, zeros16)


def _sc_aggregate(g, src2d, dst2d, zeros128):
    @functools.partial(
        pl.kernel,
        out_type=jax.ShapeDtypeStruct((NC, NPAD, D), jnp.float32),
        mesh=_vector_mesh,
        scratch_types=[
            pltpu.VMEM((HALF, C), jnp.int32),
            pltpu.VMEM((HALF, C), jnp.int32),
            pltpu.VMEM((C, D), jnp.float32),
            pltpu.VMEM((C, D), jnp.float32),
            pltpu.VMEM_SHARED((NPAD, D), jnp.float32),
            pltpu.SemaphoreType.DMA,
            pltpu.SemaphoreType.DMA,
            pltpu.SemaphoreType.DMA,
            pltpu.SemaphoreType.DMA,
        ],
    )
    def k(g_hbm, src_hbm, dst_hbm, z_hbm, out_hbm, srcv, dstv, buf0, buf1,
          acc, semg0, semg1, sems0, sems1):
        c = lax.axis_index("c")
        s = lax.axis_index("s")
        wid = s * NC + c

        pltpu.sync_copy(z_hbm.at[pl.ds(s * STRIPE, STRIPE)],
                        acc.at[pl.ds(s * STRIPE, STRIPE)])
        plsc.subcore_barrier()

        @pl.loop(0, 2)
        def _half(h):
            base = wid * WCH + h * HALF
            pltpu.sync_copy(src_hbm.at[pl.ds(base, HALF)], srcv)
            pltpu.sync_copy(dst_hbm.at[pl.ds(base, HALF)], dstv)
            pltpu.async_copy(g_hbm.at[srcv.at[0]], buf0, semg0)
            pltpu.async_copy(g_hbm.at[srcv.at[1]], buf1, semg1)

            @pl.loop(0, HALF // 2)
            def _acc(t):
                j0 = 2 * t
                j1 = 2 * t + 1
                j2 = jnp.minimum(j0 + 2, HALF - 1)
                j3 = jnp.minimum(j1 + 2, HALF - 1)
                pltpu.make_async_copy(g_hbm.at[srcv.at[j0]], buf0, semg0).wait()
                pltpu.async_copy(buf0, acc.at[dstv.at[j0]], sems0, add=True)
                pltpu.make_async_copy(g_hbm.at[srcv.at[j1]], buf1, semg1).wait()
                pltpu.async_copy(buf1, acc.at[dstv.at[j1]], sems1, add=True)
                pltpu.make_async_copy(buf0, acc.at[dstv.at[j0]], sems0).wait()
                pltpu.async_copy(g_hbm.at[srcv.at[j2]], buf0, semg0)
                pltpu.make_async_copy(buf1, acc.at[dstv.at[j1]], sems1).wait()
                pltpu.async_copy(g_hbm.at[srcv.at[j3]], buf1, semg1)

            pltpu.make_async_copy(g_hbm.at[srcv.at[0]], buf0, semg0).wait()
            pltpu.make_async_copy(g_hbm.at[srcv.at[0]], buf1, semg1).wait()

        plsc.subcore_barrier()
        pltpu.sync_copy(acc.at[pl.ds(s * STRIPE, STRIPE)],
                        out_hbm.at[c].at[pl.ds(s * STRIPE, STRIPE)])

    return k(g, src2d, dst2d, zeros128)


def _dinv_block(h_ref):
    deg = h_ref[0, :, 0:1] + h_ref[1, :, 0:1] + 1.0
    return lax.rsqrt(deg)


def _tc_pre(xp, W1, hist):
    def body(x_ref, w_ref, h_ref, o_ref):
        dinv = _dinv_block(h_ref)
        o_ref[...] = dinv * jnp.dot(x_ref[...], w_ref[...],
                                    preferred_element_type=jnp.float32,
                                    precision=lax.Precision.HIGHEST)

    return pl.pallas_call(
        body,
        grid=(NPAD // BT,),
        in_specs=[
            pl.BlockSpec((BT, D), lambda i: (i, 0)),
            pl.BlockSpec((D, D), lambda i: (0, 0)),
            pl.BlockSpec((NC, BT, 16), lambda i: (0, i, 0)),
        ],
        out_specs=pl.BlockSpec((BT, D), lambda i: (i, 0)),
        out_shape=jax.ShapeDtypeStruct((NPAD, D), jnp.float32),
    )(xp, W1, hist)


def _tc_mid(acc1, g1, hist, b1, W2):
    def body(a_ref, g_ref, h_ref, b_ref, w_ref, o_ref):
        dinv = _dinv_block(h_ref)
        y = dinv * (a_ref[0] + a_ref[1] + g_ref[...]) + b_ref[...]
        y = jnp.maximum(y, 0.0)
        o_ref[...] = dinv * jnp.dot(y, w_ref[...],
                                    preferred_element_type=jnp.float32,
                                    precision=lax.Precision.HIGHEST)

    return pl.pallas_call(
        body,
        grid=(NPAD // BT,),
        in_specs=[
            pl.BlockSpec((NC, BT, D), lambda i: (0, i, 0)),
            pl.BlockSpec((BT, D), lambda i: (i, 0)),
            pl.BlockSpec((NC, BT, 16), lambda i: (0, i, 0)),
            pl.BlockSpec((1, D), lambda i: (0, 0)),
            pl.BlockSpec((D, D), lambda i: (0, 0)),
        ],
        out_specs=pl.BlockSpec((BT, D), lambda i: (i, 0)),
        out_shape=jax.ShapeDtypeStruct((NPAD, D), jnp.float32),
    )(acc1, g1, hist, b1, W2)


def _tc_post(acc2, g2, hist, b2):
    def body(a_ref, g_ref, h_ref, b_ref, o_ref):
        dinv = _dinv_block(h_ref)
        o_ref[...] = dinv * (a_ref[0] + a_ref[1] + g_ref[...]) + b_ref[...]

    return pl.pallas_call(
        body,
        grid=(NPAD // BT,),
        in_specs=[
            pl.BlockSpec((NC, BT, D), lambda i: (0, i, 0)),
            pl.BlockSpec((BT, D), lambda i: (i, 0)),
            pl.BlockSpec((NC, BT, 16), lambda i: (0, i, 0)),
            pl.BlockSpec((1, D), lambda i: (0, 0)),
        ],
        out_specs=pl.BlockSpec((BT, D), lambda i: (i, 0)),
        out_shape=jax.ShapeDtypeStruct((NPAD, D), jnp.float32),
    )(acc2, g2, hist, b2)


def kernel(x, edge_index, W1, b1, W2, b2):
    src = edge_index[0].astype(jnp.int32)
    dst = edge_index[1].astype(jnp.int32)

    pad = PE - E
    dummy = N + (jnp.arange(pad, dtype=jnp.int32) % (NPAD - N))
    src2d = jnp.concatenate([src, dummy]).reshape(PE // C, C)
    dst2d = jnp.concatenate([dst, dummy]).reshape(PE // C, C)

    zeros16 = jnp.zeros((NPAD, 16), jnp.float32)
    zeros128 = jnp.zeros((NPAD, D), jnp.float32)
    xp = jnp.concatenate([x, jnp.zeros((NPAD - N, D), jnp.float32)])
    b1r = b1.reshape(1, D)
    b2r = b2.reshape(1, D)

    hist = _sc_degree(dst2d, zeros16)
    g1 = _tc_pre(xp, W1, hist)
    acc1 = _sc_aggregate(g1, src2d, dst2d, zeros128)
    g2 = _tc_mid(acc1, g1, hist, b1r, W2)
    acc2 = _sc_aggregate(g2, src2d, dst2d, zeros128)
    z = _tc_post(acc2, g2, hist, b2r)
    return z[:N]


# speedup vs baseline: 1.2431x; 1.2431x over previous
import functools

import jax
import jax.numpy as jnp
from jax import lax
from jax.experimental import pallas as pl
from jax.experimental.pallas import tpu as pltpu
from jax.experimental.pallas import tpu_sc as plsc

N = 10000
E = 320000
D = 128

NC = 2
NS = 16
NW = NC * NS

C = 128
WCH = 80
HALF = WCH // 2
PE = NW * WCH * C
NPAD = 10240
STRIPE = NPAD // NS

BT = 1024

_vector_mesh = plsc.VectorSubcoreMesh(core_axis_name="c", subcore_axis_name="s")


def _sc_degree(dst2d, zeros16):
    @functools.partial(
        pl.kernel,
        out_type=jax.ShapeDtypeStruct((NC, NPAD, 16), jnp.float32),
        mesh=_vector_mesh,
        scratch_types=[
            pltpu.VMEM((WCH, C), jnp.int32),
            pltpu.VMEM((C, 16), jnp.float32),
            pltpu.VMEM_SHARED((NPAD, 16), jnp.float32),
            pltpu.SemaphoreType.DMA,
        ],
    )
    def k(dst_hbm, z_hbm, out_hbm, dstv, ones, hist, sem):
        c = lax.axis_index("c")
        s = lax.axis_index("s")
        wid = s * NC + c

        @pl.loop(0, C)
        def _fill(i):
            ones[i, :] = jnp.full((16,), 1.0, jnp.float32)

        pltpu.sync_copy(z_hbm.at[pl.ds(s * STRIPE, STRIPE)],
                        hist.at[pl.ds(s * STRIPE, STRIPE)])
        pltpu.sync_copy(dst_hbm.at[pl.ds(wid * WCH, WCH)], dstv)
        plsc.subcore_barrier()

        @pl.loop(0, WCH)
        def _acc(j):
            pltpu.sync_copy(ones, hist.at[dstv.at[j]], add=True)

        plsc.subcore_barrier()
        pltpu.sync_copy(hist.at[pl.ds(s * STRIPE, STRIPE)],
                        out_hbm.at[c].at[pl.ds(s * STRIPE, STRIPE)])

    return k(dst2d, zeros16)


def _sc_aggregate(g, src2d, dst2d, zeros128):
    @functools.partial(
        pl.kernel,
        out_type=jax.ShapeDtypeStruct((NC, NPAD, D), jnp.float32),
        mesh=_vector_mesh,
        scratch_types=[
            pltpu.VMEM((HALF, C), jnp.int32),
            pltpu.VMEM((HALF, C), jnp.int32),
            pltpu.VMEM((C, D), jnp.float32),
            pltpu.VMEM((C, D), jnp.float32),
            pltpu.VMEM_SHARED((NPAD, D), jnp.float32),
            pltpu.SemaphoreType.DMA,
            pltpu.SemaphoreType.DMA,
        ],
    )
    def k(g_hbm, src_hbm, dst_hbm, z_hbm, out_hbm, srcv, dstv, buf0, buf1,
          acc, semg0, semg1):
        c = lax.axis_index("c")
        s = lax.axis_index("s")
        wid = s * NC + c

        pltpu.sync_copy(z_hbm.at[pl.ds(s * STRIPE, STRIPE)],
                        acc.at[pl.ds(s * STRIPE, STRIPE)])
        plsc.subcore_barrier()

        @pl.loop(0, 2)
        def _half(h):
            base = wid * WCH + h * HALF
            pltpu.sync_copy(src_hbm.at[pl.ds(base, HALF)], srcv)
            pltpu.sync_copy(dst_hbm.at[pl.ds(base, HALF)], dstv)
            pltpu.async_copy(g_hbm.at[srcv.at[0]], buf0, semg0)

            @pl.loop(0, HALF // 2)
            def _acc(t):
                j0 = 2 * t
                j1 = 2 * t + 1
                j2 = jnp.minimum(j0 + 2, HALF - 1)
                pltpu.async_copy(g_hbm.at[srcv.at[j1]], buf1, semg1)
                pltpu.make_async_copy(g_hbm.at[srcv.at[j0]], buf0, semg0).wait()
                pltpu.sync_copy(buf0, acc.at[dstv.at[j0]], add=True)
                pltpu.async_copy(g_hbm.at[srcv.at[j2]], buf0, semg0)
                pltpu.make_async_copy(g_hbm.at[srcv.at[j1]], buf1, semg1).wait()
                pltpu.sync_copy(buf1, acc.at[dstv.at[j1]], add=True)

            pltpu.make_async_copy(g_hbm.at[srcv.at[0]], buf0, semg0).wait()

        plsc.subcore_barrier()
        pltpu.sync_copy(acc.at[pl.ds(s * STRIPE, STRIPE)],
                        out_hbm.at[c].at[pl.ds(s * STRIPE, STRIPE)])

    return k(g, src2d, dst2d, zeros128)


def _dinv_block(h_ref):
    deg = h_ref[0, :, 0:1] + h_ref[1, :, 0:1] + 1.0
    return lax.rsqrt(deg)


def _tc_matmul(xp, W1):
    def body(x_ref, w_ref, o_ref):
        o_ref[...] = jnp.dot(x_ref[...], w_ref[...],
                             preferred_element_type=jnp.float32,
                             precision=lax.Precision.HIGHEST)

    return pl.pallas_call(
        body,
        grid=(NPAD // BT,),
        in_specs=[
            pl.BlockSpec((BT, D), lambda i: (i, 0)),
            pl.BlockSpec((D, D), lambda i: (0, 0)),
        ],
        out_specs=pl.BlockSpec((BT, D), lambda i: (i, 0)),
        out_shape=jax.ShapeDtypeStruct((NPAD, D), jnp.float32),
    )(xp, W1)


def _tc_scale(h1, hist):
    def body(x_ref, h_ref, o_ref):
        o_ref[...] = _dinv_block(h_ref) * x_ref[...]

    return pl.pallas_call(
        body,
        grid=(NPAD // BT,),
        in_specs=[
            pl.BlockSpec((BT, D), lambda i: (i, 0)),
            pl.BlockSpec((NC, BT, 16), lambda i: (0, i, 0)),
        ],
        out_specs=pl.BlockSpec((BT, D), lambda i: (i, 0)),
        out_shape=jax.ShapeDtypeStruct((NPAD, D), jnp.float32),
    )(h1, hist)


def _tc_mid(acc1, g1, hist, b1, W2):
    def body(a_ref, g_ref, h_ref, b_ref, w_ref, o_ref):
        dinv = _dinv_block(h_ref)
        y = dinv * (a_ref[0] + a_ref[1] + g_ref[...]) + b_ref[...]
        y = jnp.maximum(y, 0.0)
        o_ref[...] = dinv * jnp.dot(y, w_ref[...],
                                    preferred_element_type=jnp.float32,
                                    precision=lax.Precision.HIGHEST)

    return pl.pallas_call(
        body,
        grid=(NPAD // BT,),
        in_specs=[
            pl.BlockSpec((NC, BT, D), lambda i: (0, i, 0)),
            pl.BlockSpec((BT, D), lambda i: (i, 0)),
            pl.BlockSpec((NC, BT, 16), lambda i: (0, i, 0)),
            pl.BlockSpec((1, D), lambda i: (0, 0)),
            pl.BlockSpec((D, D), lambda i: (0, 0)),
        ],
        out_specs=pl.BlockSpec((BT, D), lambda i: (i, 0)),
        out_shape=jax.ShapeDtypeStruct((NPAD, D), jnp.float32),
    )(acc1, g1, hist, b1, W2)


def _tc_post(acc2, g2, hist, b2):
    def body(a_ref, g_ref, h_ref, b_ref, o_ref):
        dinv = _dinv_block(h_ref)
        o_ref[...] = dinv * (a_ref[0] + a_ref[1] + g_ref[...]) + b_ref[...]

    return pl.pallas_call(
        body,
        grid=(NPAD // BT,),
        in_specs=[
            pl.BlockSpec((NC, BT, D), lambda i: (0, i, 0)),
            pl.BlockSpec((BT, D), lambda i: (i, 0)),
            pl.BlockSpec((NC, BT, 16), lambda i: (0, i, 0)),
            pl.BlockSpec((1, D), lambda i: (0, 0)),
        ],
        out_specs=pl.BlockSpec((BT, D), lambda i: (i, 0)),
        out_shape=jax.ShapeDtypeStruct((NPAD, D), jnp.float32),
    )(acc2, g2, hist, b2)


def kernel(x, edge_index, W1, b1, W2, b2):
    src = edge_index[0].astype(jnp.int32)
    dst = edge_index[1].astype(jnp.int32)

    pad = PE - E
    dummy = N + (jnp.arange(pad, dtype=jnp.int32) % (NPAD - N))
    src2d = jnp.concatenate([src, dummy]).reshape(PE // C, C)
    dst2d = jnp.concatenate([dst, dummy]).reshape(PE // C, C)

    zeros16 = jnp.zeros((NPAD, 16), jnp.float32)
    zeros128 = jnp.zeros((NPAD, D), jnp.float32)
    xp = jnp.concatenate([x, jnp.zeros((NPAD - N, D), jnp.float32)])
    b1r = b1.reshape(1, D)
    b2r = b2.reshape(1, D)

    hist = _sc_degree(dst2d, zeros16)
    h1 = _tc_matmul(xp, W1)
    g1 = _tc_scale(h1, hist)
    acc1 = _sc_aggregate(g1, src2d, dst2d, zeros128)
    g2 = _tc_mid(acc1, g1, hist, b1r, W2)
    acc2 = _sc_aggregate(g2, src2d, dst2d, zeros128)
    z = _tc_post(acc2, g2, hist, b2r)
    return z[:N]


# speedup vs baseline: 1.2568x; 1.0110x over previous
import functools

import jax
import jax.numpy as jnp
from jax import lax
from jax.experimental import pallas as pl
from jax.experimental.pallas import tpu as pltpu
from jax.experimental.pallas import tpu_sc as plsc

N = 10000
E = 320000
D = 128

NC = 2
NS = 16
NW = NC * NS

C = 128
WCH = 80
HALF = WCH // 2
PE = NW * WCH * C
NPAD = 10240
STRIPE = NPAD // NS

BT = 1000

_vector_mesh = plsc.VectorSubcoreMesh(core_axis_name="c", subcore_axis_name="s")


def _sc_degree(dst2d, zeros16):
    @functools.partial(
        pl.kernel,
        out_type=jax.ShapeDtypeStruct((NC, NPAD, 16), jnp.float32),
        mesh=_vector_mesh,
        scratch_types=[
            pltpu.VMEM((WCH, C), jnp.int32),
            pltpu.VMEM((C, 16), jnp.float32),
            pltpu.VMEM_SHARED((NPAD, 16), jnp.float32),
            pltpu.SemaphoreType.DMA,
        ],
    )
    def k(dst_hbm, z_hbm, out_hbm, dstv, ones, hist, sem):
        c = lax.axis_index("c")
        s = lax.axis_index("s")
        wid = s * NC + c

        @pl.loop(0, C)
        def _fill(i):
            ones[i, :] = jnp.full((16,), 1.0, jnp.float32)

        pltpu.sync_copy(z_hbm.at[pl.ds(s * STRIPE, STRIPE)],
                        hist.at[pl.ds(s * STRIPE, STRIPE)])
        pltpu.sync_copy(dst_hbm.at[pl.ds(wid * WCH, WCH)], dstv)
        plsc.subcore_barrier()

        @pl.loop(0, WCH)
        def _acc(j):
            pltpu.sync_copy(ones, hist.at[dstv.at[j]], add=True)

        plsc.subcore_barrier()
        pltpu.sync_copy(hist.at[pl.ds(s * STRIPE, STRIPE)],
                        out_hbm.at[c].at[pl.ds(s * STRIPE, STRIPE)])

    return k(dst2d, zeros16)


def _sc_aggregate(g, src2d, dst2d, zeros128):
    @functools.partial(
        pl.kernel,
        out_type=jax.ShapeDtypeStruct((NC, NPAD, D), jnp.float32),
        mesh=_vector_mesh,
        scratch_types=[
            pltpu.VMEM((HALF, C), jnp.int32),
            pltpu.VMEM((HALF, C), jnp.int32),
            pltpu.VMEM((C, D), jnp.float32),
            pltpu.VMEM((C, D), jnp.float32),
            pltpu.VMEM_SHARED((NPAD, D), jnp.float32),
            pltpu.SemaphoreType.DMA,
            pltpu.SemaphoreType.DMA,
        ],
    )
    def k(g_hbm, src_hbm, dst_hbm, z_hbm, out_hbm, srcv, dstv, buf0, buf1,
          acc, semg0, semg1):
        c = lax.axis_index("c")
        s = lax.axis_index("s")
        wid = s * NC + c

        pltpu.sync_copy(z_hbm.at[pl.ds(s * STRIPE, STRIPE)],
                        acc.at[pl.ds(s * STRIPE, STRIPE)])
        plsc.subcore_barrier()

        @pl.loop(0, 2)
        def _half(h):
            base = wid * WCH + h * HALF
            pltpu.sync_copy(src_hbm.at[pl.ds(base, HALF)], srcv)
            pltpu.sync_copy(dst_hbm.at[pl.ds(base, HALF)], dstv)
            pltpu.async_copy(g_hbm.at[srcv.at[0]], buf0, semg0)

            @pl.loop(0, HALF // 2)
            def _acc(t):
                j0 = 2 * t
                j1 = 2 * t + 1
                j2 = jnp.minimum(j0 + 2, HALF - 1)
                pltpu.async_copy(g_hbm.at[srcv.at[j1]], buf1, semg1)
                pltpu.make_async_copy(g_hbm.at[srcv.at[j0]], buf0, semg0).wait()
                pltpu.sync_copy(buf0, acc.at[dstv.at[j0]], add=True)
                pltpu.async_copy(g_hbm.at[srcv.at[j2]], buf0, semg0)
                pltpu.make_async_copy(g_hbm.at[srcv.at[j1]], buf1, semg1).wait()
                pltpu.sync_copy(buf1, acc.at[dstv.at[j1]], add=True)

            pltpu.make_async_copy(g_hbm.at[srcv.at[0]], buf0, semg0).wait()

        plsc.subcore_barrier()
        pltpu.sync_copy(acc.at[pl.ds(s * STRIPE, STRIPE)],
                        out_hbm.at[c].at[pl.ds(s * STRIPE, STRIPE)])

    return k(g, src2d, dst2d, zeros128)


def _dinv_block(h_ref):
    deg = h_ref[0, :, 0:1] + h_ref[1, :, 0:1] + 1.0
    return lax.rsqrt(deg)


def _tc_matmul(x, W1):
    def body(x_ref, w_ref, o_ref):
        o_ref[...] = jnp.dot(x_ref[...], w_ref[...],
                             preferred_element_type=jnp.float32,
                             precision=lax.Precision.HIGHEST)

    return pl.pallas_call(
        body,
        grid=(N // BT,),
        in_specs=[
            pl.BlockSpec((BT, D), lambda i: (i, 0)),
            pl.BlockSpec((D, D), lambda i: (0, 0)),
        ],
        out_specs=pl.BlockSpec((BT, D), lambda i: (i, 0)),
        out_shape=jax.ShapeDtypeStruct((N, D), jnp.float32),
    )(x, W1)


def _tc_scale(h1, hist):
    def body(x_ref, h_ref, o_ref):
        o_ref[...] = _dinv_block(h_ref) * x_ref[...]

    return pl.pallas_call(
        body,
        grid=(N // BT,),
        in_specs=[
            pl.BlockSpec((BT, D), lambda i: (i, 0)),
            pl.BlockSpec((NC, BT, 16), lambda i: (0, i, 0)),
        ],
        out_specs=pl.BlockSpec((BT, D), lambda i: (i, 0)),
        out_shape=jax.ShapeDtypeStruct((N, D), jnp.float32),
    )(h1, hist)


def _tc_mid(acc1, g1, hist, b1, W2):
    def body(a_ref, g_ref, h_ref, b_ref, w_ref, o_ref):
        dinv = _dinv_block(h_ref)
        y = dinv * (a_ref[0] + a_ref[1] + g_ref[...]) + b_ref[...]
        y = jnp.maximum(y, 0.0)
        o_ref[...] = dinv * jnp.dot(y, w_ref[...],
                                    preferred_element_type=jnp.float32,
                                    precision=lax.Precision.HIGHEST)

    return pl.pallas_call(
        body,
        grid=(N // BT,),
        in_specs=[
            pl.BlockSpec((NC, BT, D), lambda i: (0, i, 0)),
            pl.BlockSpec((BT, D), lambda i: (i, 0)),
            pl.BlockSpec((NC, BT, 16), lambda i: (0, i, 0)),
            pl.BlockSpec((1, D), lambda i: (0, 0)),
            pl.BlockSpec((D, D), lambda i: (0, 0)),
        ],
        out_specs=pl.BlockSpec((BT, D), lambda i: (i, 0)),
        out_shape=jax.ShapeDtypeStruct((N, D), jnp.float32),
    )(acc1, g1, hist, b1, W2)


def _tc_post(acc2, g2, hist, b2):
    def body(a_ref, g_ref, h_ref, b_ref, o_ref):
        dinv = _dinv_block(h_ref)
        o_ref[...] = dinv * (a_ref[0] + a_ref[1] + g_ref[...]) + b_ref[...]

    return pl.pallas_call(
        body,
        grid=(N // BT,),
        in_specs=[
            pl.BlockSpec((NC, BT, D), lambda i: (0, i, 0)),
            pl.BlockSpec((BT, D), lambda i: (i, 0)),
            pl.BlockSpec((NC, BT, 16), lambda i: (0, i, 0)),
            pl.BlockSpec((1, D), lambda i: (0, 0)),
        ],
        out_specs=pl.BlockSpec((BT, D), lambda i: (i, 0)),
        out_shape=jax.ShapeDtypeStruct((N, D), jnp.float32),
    )(acc2, g2, hist, b2)


def kernel(x, edge_index, W1, b1, W2, b2):
    src = edge_index[0].astype(jnp.int32)
    dst = edge_index[1].astype(jnp.int32)

    pad = PE - E
    ar = jnp.arange(pad, dtype=jnp.int32)
    src2d = jnp.concatenate([src, ar % N]).reshape(PE // C, C)
    dst2d = jnp.concatenate([dst, N + (ar % (NPAD - N))]).reshape(PE // C, C)

    zeros16 = jnp.zeros((NPAD, 16), jnp.float32)
    zeros128 = jnp.zeros((NPAD, D), jnp.float32)
    b1r = b1.reshape(1, D)
    b2r = b2.reshape(1, D)

    hist = _sc_degree(dst2d, zeros16)
    h1 = _tc_matmul(x, W1)
    g1 = _tc_scale(h1, hist)
    acc1 = _sc_aggregate(g1, src2d, dst2d, zeros128)
    g2 = _tc_mid(acc1, g1, hist, b1r, W2)
    acc2 = _sc_aggregate(g2, src2d, dst2d, zeros128)
    return _tc_post(acc2, g2, hist, b2r)
